# baseline (speedup 1.0000x reference)
import functools
import math

import jax
import jax.numpy as jnp
from jax import lax
from jax.experimental import pallas as pl
from jax.experimental.pallas import tpu as pltpu
from jax.experimental.pallas import tpu_sc as plsc

N_ENT = 10000
N_REL = 200
N_QREL = 50
D = 128
NHEAD = 4
DH = D // NHEAD
NLAYERS = 2

_NSUB = 16
_NCORE = 2
_SUB = 80
_QS = 8
_HS = _QS // 2
_CE = _SUB * _HS
_CH4 = 2
_W = 128
_CE4 = _W * _CH4
_ACC = 10240
_JUNK = N_ENT


def _ln_val(x, s, b):
    m = jnp.mean(x, -1, keepdims=True)
    v = jnp.mean((x - m) * (x - m), -1, keepdims=True)
    return (x - m) / jnp.sqrt(v + 1e-5) * s + b


def _t1_body(x_ref, mb_ref, wq, bq, wk, bk, wv, bv, wo, bo,
             w1, b1, w2, b2, l1s, l1b, l2s, l2b, out_ref):
    nb = x_ref.shape[0]
    scale = 1.0 / math.sqrt(DH)
    for b in range(nb):
        x = x_ref[b]
        mrow = mb_ref[b]
        for l in range(NLAYERS):
            q = jnp.dot(x, wq[l], preferred_element_type=jnp.float32) + bq[l][None, :]
            k = jnp.dot(x, wk[l], preferred_element_type=jnp.float32) + bk[l][None, :]
            v = jnp.dot(x, wv[l], preferred_element_type=jnp.float32) + bv[l][None, :]
            heads = []
            for h in range(NHEAD):
                sl = slice(h * DH, (h + 1) * DH)
                qh, kh, vh = q[:, sl], k[:, sl], v[:, sl]
                lg = lax.dot_general(qh, kh, (((1,), (1,)), ((), ())),
                                     preferred_element_type=jnp.float32)
                lg = lg * scale + mrow[None, :]
                mx = jnp.max(lg, -1, keepdims=True)
                e = jnp.exp(lg - mx)
                a = e / jnp.sum(e, -1, keepdims=True)
                heads.append(jnp.dot(a, vh, preferred_element_type=jnp.float32))
            o = jnp.concatenate(heads, axis=1)
            o = jnp.dot(o, wo[l], preferred_element_type=jnp.float32) + bo[l][None, :]
            x = _ln_val(x + o, l1s[l][None, :], l1b[l][None, :])
            ff = jnp.maximum(jnp.dot(x, w1[l], preferred_element_type=jnp.float32)
                             + b1[l][None, :], 0.0)
            ff = jnp.dot(ff, w2[l], preferred_element_type=jnp.float32) + b2[l][None, :]
            x = _ln_val(x + ff, l2s[l][None, :], l2b[l][None, :])
        out_ref[pl.ds(b, 1), :] = jnp.mean(x, axis=0)[None, :]


def _t1_call(stk, maskbias, tw):
    nb = stk.shape[0]
    args = [stk, maskbias]
    for name in ('Wq', 'bq', 'Wk', 'bk', 'Wv', 'bv', 'Wo', 'bo',
                 'W1', 'b1', 'W2', 'b2', 'ln1_s', 'ln1_b', 'ln2_s', 'ln2_b'):
        args.append(jnp.stack([p[name] for p in tw]))
    return pl.pallas_call(
        _t1_body,
        out_shape=jax.ShapeDtypeStruct((nb, D), jnp.float32),
    )(*args)


_RB = 2000


def _t2_body(h0_ref, agg_ref, hid_ref, w0_ref, w1_ref, b_ref,
             ls_ref, lb_ref, q_ref, out_ref):
    i = pl.program_id(0)
    a = agg_ref[...]
    x = jnp.concatenate([a[0], a[1]], axis=-1)
    rows = lax.broadcasted_iota(jnp.int32, (_RB, 1), 0) + i * _RB
    for b in range(q_ref.shape[0]):
        mask = (rows == h0_ref[b]).astype(jnp.float32)
        x = x + mask * q_ref[b][None, :]
    h = hid_ref[...]
    y = (jnp.dot(x, w0_ref[...], preferred_element_type=jnp.float32)
         + jnp.dot(h, w1_ref[...], preferred_element_type=jnp.float32)
         + b_ref[0][None, :])
    y = _ln_val(y, ls_ref[0][None, :], lb_ref[0][None, :])
    out_ref[...] = jnp.maximum(y, 0.0)


def _t2_call(agg2, hid, w, bvec, ln_s, ln_b, query, h0rows):
    bn = agg2.shape[1]
    grid = (bn // _RB,)
    return pl.pallas_call(
        _t2_body,
        grid=grid,
        in_specs=[
            pl.BlockSpec(memory_space=pltpu.SMEM),
            pl.BlockSpec((2, _RB, D // 2), lambda i: (0, i, 0)),
            pl.BlockSpec((_RB, D), lambda i: (i, 0)),
            pl.BlockSpec((D, D), lambda i: (0, 0)),
            pl.BlockSpec((D, D), lambda i: (0, 0)),
            pl.BlockSpec((1, D), lambda i: (0, 0)),
            pl.BlockSpec((1, D), lambda i: (0, 0)),
            pl.BlockSpec((1, D), lambda i: (0, 0)),
            pl.BlockSpec((4, D), lambda i: (0, 0)),
        ],
        out_specs=pl.BlockSpec((_RB, D), lambda i: (i, 0)),
        out_shape=jax.ShapeDtypeStruct((bn, D), jnp.float32),
    )(h0rows, agg2, hid, w[:D], w[D:], bvec[None, :], ln_s[None, :],
      ln_b[None, :], query)


def _t3_body(f_ref, w1_ref, b1_ref, w2_ref, b2_ref, out_ref):
    hm = jnp.maximum(
        jnp.dot(f_ref[...], w1_ref[...], preferred_element_type=jnp.float32)
        + b1_ref[0][None, :], 0.0)
    s = jnp.sum(hm * w2_ref[...][:, 0][None, :], axis=-1, keepdims=True)
    out_ref[...] = s + b2_ref[0][None, :]


def _t3_call(feat, w1, b1, w2, b2):
    rows = feat.shape[0]
    return pl.pallas_call(
        _t3_body,
        out_shape=jax.ShapeDtypeStruct((rows, 1), jnp.float32),
    )(feat, w1, b1[None, :], w2, b2[None, :])


_D2 = D // 2


def _sc_body(hid2, relw2, eidx, agg2,
             ia, ib, ha, ra, hb, rb, acc, sema, semb, semsa, semsb):
    c = lax.axis_index("c")
    s = lax.axis_index("s")
    stripe = _ACC // _NSUB
    rows_per_tile = eidx.shape[0] // _NSUB
    chunks = rows_per_tile // _CH4
    pairs = chunks // 2

    def load_idx(ibuf, rowbase):
        pltpu.sync_copy(eidx.at[pl.ds(rowbase, _CH4)], ibuf)

    def offadd(ibuf, off, c):
        for q in range(_CH4):
            for u in range(_W // 16):
                sl = pl.ds(u * 16, 16)
                ibuf[q, 0, sl] = ibuf[q, 0, sl] * 2 + off
                ibuf[q, 2, sl] = ibuf[q, 2, sl] * 2 + c

    def fire(ibuf, hbuf, rbuf, sem):
        for q in range(_CH4):
            pltpu.async_copy(hid2.at[ibuf.at[q, 0]],
                             hbuf.at[pl.ds(q * _W, _W)], sem)
            pltpu.async_copy(relw2.at[ibuf.at[q, 2]],
                             rbuf.at[pl.ds(q * _W, _W)], sem)

    def drain(ibuf, hbuf, rbuf, sem):
        for q in range(_CH4):
            pltpu.make_async_copy(hid2.at[ibuf.at[q, 0]],
                                  hbuf.at[pl.ds(q * _W, _W)], sem).wait()
            pltpu.make_async_copy(relw2.at[ibuf.at[q, 2]],
                                  rbuf.at[pl.ds(q * _W, _W)], sem).wait()

    def mul(hbuf, rbuf):
        def mb(r, _):
            for rr in range(8):
                row = r * 8 + rr
                for u in range(_D2 // 16):
                    sl = pl.ds(u * 16, 16)
                    hbuf[row, sl] = hbuf[row, sl] * rbuf[row, sl]
            return 0
        lax.fori_loop(0, _CE4 // 8, mb, 0)

    def fire_scat(ibuf, hbuf, sem):
        for q in range(_CH4):
            pltpu.async_copy(hbuf.at[pl.ds(q * _W, _W)],
                             acc.at[ibuf.at[q, 1]], sem, add=True)

    def drain_scat(hbuf, sem):
        for q in range(_CH4):
            pltpu.make_async_copy(hid2.at[pl.ds(0, _W)],
                                  hbuf.at[pl.ds(q * _W, _W)], sem).wait()

    for b in range(4):
        bn = b * N_ENT
        off = 2 * bn + c

        def zbody(r, _):
            for u in range(_D2 // 16):
                ha[r, pl.ds(u * 16, 16)] = jnp.zeros((16,), jnp.float32)
            return 0
        lax.fori_loop(0, _CE4, zbody, 0)
        base = s * stripe
        pltpu.sync_copy(ha.at[pl.ds(0, _CE4)], acc.at[pl.ds(base, _CE4)])
        pltpu.sync_copy(ha.at[pl.ds(0, _CE4)],
                        acc.at[pl.ds(base + _CE4, _CE4)])
        pltpu.sync_copy(ha.at[pl.ds(0, stripe - 2 * _CE4)],
                        acc.at[pl.ds(base + 2 * _CE4, stripe - 2 * _CE4)])
        plsc.subcore_barrier()

        tilebase = s * rows_per_tile
        load_idx(ia, tilebase)
        offadd(ia, off, c)
        fire(ia, ha, ra, sema)

        def body(k, _):
            @pl.when(k > 0)
            def _dsb():
                drain_scat(hb, semsb)
            rbb = tilebase + (2 * k + 1) * _CH4
            load_idx(ib, rbb)
            offadd(ib, off, c)
            fire(ib, hb, rb, semb)
            drain(ia, ha, ra, sema)
            mul(ha, ra)
            fire_scat(ia, ha, semsa)

            @pl.when(k < pairs - 1)
            def _next():
                drain_scat(ha, semsa)
                rba = tilebase + (2 * k + 2) * _CH4
                load_idx(ia, rba)
                offadd(ia, off, c)
                fire(ia, ha, ra, sema)
            drain(ib, hb, rb, semb)
            mul(hb, rb)
            fire_scat(ib, hb, semsb)
            return 0
        lax.fori_loop(0, pairs, body, 0)
        drain_scat(ha, semsa)
        drain_scat(hb, semsb)
        plsc.subcore_barrier()
        tail = N_ENT - (_NSUB - 1) * stripe

        @pl.when(s < _NSUB - 1)
        def _copy_full():
            pltpu.sync_copy(acc.at[pl.ds(base, stripe)],
                            agg2.at[c, pl.ds(bn + base, stripe)])

        @pl.when(s == _NSUB - 1)
        def _copy_tail():
            pltpu.sync_copy(acc.at[pl.ds((_NSUB - 1) * stripe, tail)],
                            agg2.at[c, pl.ds(bn + (_NSUB - 1) * stripe, tail)])
        plsc.subcore_barrier()


def _sc_call(hid, relw, eidx):
    bn = hid.shape[0]
    hid2 = hid.reshape(2 * bn, _D2)
    relw2 = relw.reshape(2 * relw.shape[0], _D2)
    mesh = plsc.VectorSubcoreMesh(core_axis_name="c", subcore_axis_name="s")
    f = pl.kernel(
        _sc_body,
        out_type=jax.ShapeDtypeStruct((2, bn, _D2), jnp.float32),
        mesh=mesh,
        compiler_params=pltpu.CompilerParams(use_tc_tiling_on_sc=False),
        scratch_types=[
            pltpu.VMEM((_CH4, 3, _W), jnp.int32),
            pltpu.VMEM((_CH4, 3, _W), jnp.int32),
            pltpu.VMEM((_CE4, _D2), jnp.float32),
            pltpu.VMEM((_CE4, _D2), jnp.float32),
            pltpu.VMEM((_CE4, _D2), jnp.float32),
            pltpu.VMEM((_CE4, _D2), jnp.float32),
            pltpu.VMEM_SHARED((_ACC, _D2), jnp.float32),
            pltpu.SemaphoreType.DMA,
            pltpu.SemaphoreType.DMA,
            pltpu.SemaphoreType.DMA,
            pltpu.SemaphoreType.DMA,
        ],
    )
    return f(hid2, relw2, eidx)


_CH = 64


def _sc1_body(relw2, src2, dst2, et2, h0bc, qrow2, agg2,
              srcv, dstv, etv, mdstbuf, metbuf, h0v, qv, metv, mdstv,
              rbuf, zbuf, acc, sem):
    c = lax.axis_index("c")
    s = lax.axis_index("s")
    stripe = _ACC // _NSUB
    rows_per_tile = src2.shape[0] // _NSUB
    chunks = rows_per_tile // _CH
    lane = jax.lax.broadcasted_iota(jnp.int32, (16,), 0)

    def zfill(r, _):
        for u in range(_D2 // 16):
            zbuf[r, pl.ds(u * 16, 16)] = jnp.zeros((16,), jnp.float32)
        return 0
    lax.fori_loop(0, 320, zfill, 0)

    for b in range(4):
        bn = b * N_ENT

        pltpu.sync_copy(zbuf, acc.at[pl.ds(s * stripe, 320)])
        pltpu.sync_copy(zbuf, acc.at[pl.ds(s * stripe + 320, 320)])
        plsc.subcore_barrier()

        pltpu.sync_copy(h0bc.at[b], h0v)
        pltpu.sync_copy(qrow2.at[2 * b + c], qv)
        h0vec = h0v[...]
        qvv = [qv[pl.ds(u * 16, 16)] for u in range(_D2 // 16)]

        for ch in range(chunks):
            rowbase = s * rows_per_tile + ch * _CH
            pltpu.sync_copy(src2.at[pl.ds(rowbase, _CH)], srcv)
            pltpu.sync_copy(dst2.at[pl.ds(rowbase, _CH)], dstv)
            pltpu.sync_copy(et2.at[pl.ds(rowbase, _CH)], etv)

            def scan_row(q, cnt):
                for u in range(_SUB // 16):
                    sl = pl.ds(u * 16, 16)
                    s16 = srcv[q, sl]
                    m = s16 == h0vec
                    pop = plsc.all_reduce_population_count(m)[0]

                    @pl.when(pop > 0)
                    def _store():
                        plsc.store_compressed(mdstbuf.at[pl.ds(cnt, 16)],
                                              dstv[q, sl], mask=m)
                        plsc.store_compressed(metbuf.at[pl.ds(cnt, 16)],
                                              etv[q, sl], mask=m)
                    cnt = cnt + pop
                return cnt
            cnt = lax.fori_loop(0, _CH, scan_row, jnp.int32(0))

            def pgroup(k, _):
                valid = lane + k * 16 < cnt
                met16 = metbuf[pl.ds(k * 16, 16)]
                md16 = mdstbuf[pl.ds(k * 16, 16)]
                metv[...] = jnp.where(valid, met16 * 2 + c, 0)
                mdstv[...] = jnp.where(valid, md16, _JUNK)
                pltpu.async_copy(relw2.at[metv], rbuf, sem).wait()
                for r in range(16):
                    for u in range(_D2 // 16):
                        sl = pl.ds(u * 16, 16)
                        rbuf[r, sl] = rbuf[r, sl] * qvv[u]
                pltpu.sync_copy(rbuf, acc.at[mdstv], add=True)
                return 0
            lax.fori_loop(0, lax.shift_right_logical(cnt + 15, 4), pgroup, 0)
            @pl.when(cnt > 0)
            def _rezero():
                for r in range(16):
                    for u in range(_D2 // 16):
                        rbuf[r, pl.ds(u * 16, 16)] = jnp.zeros((16,),
                                                               jnp.float32)

        plsc.subcore_barrier()
        tail = N_ENT - (_NSUB - 1) * stripe

        @pl.when(s < _NSUB - 1)
        def _copy_full():
            pltpu.sync_copy(acc.at[pl.ds(s * stripe, stripe)],
                            agg2.at[c, pl.ds(bn + s * stripe, stripe)])

        @pl.when(s == _NSUB - 1)
        def _copy_tail():
            pltpu.sync_copy(acc.at[pl.ds((_NSUB - 1) * stripe, tail)],
                            agg2.at[c, pl.ds(bn + (_NSUB - 1) * stripe, tail)])
        plsc.subcore_barrier()


def _sc1_call(relw, src2, dst2, et2, h0bc, query):
    bn = 4 * N_ENT
    relw2 = relw.reshape(2 * relw.shape[0], _D2)
    qrow2 = query.reshape(2 * query.shape[0], _D2)
    mesh = plsc.VectorSubcoreMesh(core_axis_name="c", subcore_axis_name="s")
    maxm = _CH * _SUB + 16
    f = pl.kernel(
        _sc1_body,
        out_type=jax.ShapeDtypeStruct((2, bn, _D2), jnp.float32),
        mesh=mesh,
        compiler_params=pltpu.CompilerParams(use_tc_tiling_on_sc=False,
                                             needs_layout_passes=False),
        scratch_types=[
            pltpu.VMEM((_CH, _SUB), jnp.int32),
            pltpu.VMEM((_CH, _SUB), jnp.int32),
            pltpu.VMEM((_CH, _SUB), jnp.int32),
            pltpu.VMEM((maxm,), jnp.int32),
            pltpu.VMEM((maxm,), jnp.int32),
            pltpu.VMEM((16,), jnp.int32),
            pltpu.VMEM((_D2,), jnp.float32),
            pltpu.VMEM((16,), jnp.int32),
            pltpu.VMEM((16,), jnp.int32),
            pltpu.VMEM((16, _D2), jnp.float32),
            pltpu.VMEM((320, _D2), jnp.float32),
            pltpu.VMEM_SHARED((_ACC, _D2), jnp.float32),
            pltpu.SemaphoreType.DMA,
        ],
    )
    return f(relw2, src2, dst2, et2, h0bc, qrow2)


def _t2l1_body(h0_ref, agg_ref, w0_ref, w1_ref, b_ref,
               ls_ref, lb_ref, q_ref, out_ref):
    i = pl.program_id(0)
    a = agg_ref[...]
    x = jnp.concatenate([a[0], a[1]], axis=-1)
    rows = lax.broadcasted_iota(jnp.int32, (_RB, 1), 0) + i * _RB
    qw1 = jnp.dot(q_ref[...], w1_ref[...], preferred_element_type=jnp.float32)
    hcon = jnp.zeros((_RB, D), jnp.float32)
    for b in range(q_ref.shape[0]):
        mask = (rows == h0_ref[b]).astype(jnp.float32)
        x = x + mask * q_ref[b][None, :]
        hcon = hcon + mask * qw1[b][None, :]
    y = (jnp.dot(x, w0_ref[...], preferred_element_type=jnp.float32)
         + hcon + b_ref[0][None, :])
    y = _ln_val(y, ls_ref[0][None, :], lb_ref[0][None, :])
    out_ref[...] = jnp.maximum(y, 0.0)


def _t2l1_call(agg2, w, bvec, ln_s, ln_b, query, h0rows):
    bn = agg2.shape[1]
    grid = (bn // _RB,)
    return pl.pallas_call(
        _t2l1_body,
        grid=grid,
        in_specs=[
            pl.BlockSpec(memory_space=pltpu.SMEM),
            pl.BlockSpec((2, _RB, D // 2), lambda i: (0, i, 0)),
            pl.BlockSpec((D, D), lambda i: (0, 0)),
            pl.BlockSpec((D, D), lambda i: (0, 0)),
            pl.BlockSpec((1, D), lambda i: (0, 0)),
            pl.BlockSpec((1, D), lambda i: (0, 0)),
            pl.BlockSpec((1, D), lambda i: (0, 0)),
            pl.BlockSpec((4, D), lambda i: (0, 0)),
        ],
        out_specs=pl.BlockSpec((_RB, D), lambda i: (i, 0)),
        out_shape=jax.ShapeDtypeStruct((bn, D), jnp.float32),
    )(h0rows, agg2, w[:D], w[D:], bvec[None, :], ln_s[None, :],
      ln_b[None, :], query)


def kernel(params, h_index, t_index, r_index, edge_index, edge_type):
    shape = h_index.shape
    B, K = shape

    quals = r_index[:, K:]
    r_main = r_index[:, :K]
    qr_index = quals[:, 0::2]
    qe_index = quals[:, 1::2]
    qual_mask = qr_index == -1
    qr_emb = jnp.where(qual_mask[..., None], 0.0,
                       params['qual_rel_embed'][jnp.clip(qr_index, 0)])
    qe_emb = jnp.where(qual_mask[..., None], 0.0,
                       params['ent_embed'][jnp.clip(qe_index, 0)])
    r_emb = params['rel_embed'][r_main[:, 0]].reshape(-1, 1, D)
    quals_emb = jnp.concatenate([qr_emb, qe_emb], 2).reshape(
        B, 2 * qr_emb.shape[1], D)
    stk = jnp.concatenate([r_emb, quals_emb], 1)
    S = stk.shape[1]
    stk = stk + params['pos_embed'][:S][None, :, :]
    rel_mask = jnp.zeros((B, 1), dtype=bool)
    seq_mask = jnp.concatenate([rel_mask, qual_mask, qual_mask], 1)
    maskbias = jnp.where(seq_mask, -1e9, 0.0).astype(jnp.float32)

    query = _t1_call(stk, maskbias, params['transformer'])

    src = edge_index[0].astype(jnp.int32)
    dst = edge_index[1].astype(jnp.int32)
    et = edge_type.astype(jnp.int32)
    E = src.shape[0]
    unit = _NSUB * _QS * _SUB
    pad = (-E) % unit
    if pad:
        src = jnp.concatenate([src, jnp.zeros((pad,), jnp.int32)])
        dst = jnp.concatenate([dst, jnp.full((pad,), _JUNK, jnp.int32)])
        et = jnp.concatenate([et, jnp.zeros((pad,), jnp.int32)])
    ep = E + pad
    src2 = src.reshape(ep // _SUB, _SUB)
    dst2 = dst.reshape(ep // _SUB, _SUB)
    et2 = et.reshape(ep // _SUB, _SUB)
    eidx = jnp.stack([src.reshape(ep // _W, _W), dst.reshape(ep // _W, _W),
                      et.reshape(ep // _W, _W)], axis=1)

    h0rows = (jnp.arange(B, dtype=jnp.int32) * N_ENT
              + h_index[:, 0].astype(jnp.int32))
    h0bc = jnp.broadcast_to(h_index[:, :1].astype(jnp.int32), (B, 16))

    p0 = params['nbf'][0]
    agg = _sc1_call(p0['rel_w'], src2, dst2, et2, h0bc, query)
    hid = _t2l1_call(agg, p0['W'], p0['b'], p0['ln_s'], p0['ln_b'],
                     query, h0rows)
    p1 = params['nbf'][1]
    agg = _sc_call(hid, p1['rel_w'], eidx)
    hid = _t2_call(agg, hid, p1['W'], p1['b'], p1['ln_s'], p1['ln_b'],
                   query, h0rows)

    hidB = hid.reshape(B, N_ENT, D)
    feat_h = hidB[jnp.arange(B)[:, None], t_index]
    feat = jnp.concatenate(
        [feat_h, jnp.broadcast_to(query[:, None, :], (B, K, D))], -1)
    m = params['mlp']
    score = _t3_call(feat.reshape(B * K, 2 * D), m['W1'], m['b1'],
                     m['W2'], m['b2'])
    return score.reshape(shape)
